# baseline (speedup 1.0000x reference)
import functools

import jax
import jax.numpy as jnp
from jax import lax
from jax.experimental import pallas as pl
from jax.experimental.pallas import tpu as pltpu
from jax.experimental.pallas import tpu_sc as plsc

N = 10000
E = 320000
L = 7
F = 128
FM = 16
H = 128
T = 12
NG = 64
MH = 2 * (2 * F + 1)

NC = 2
NS = 16
NW = NC * NS
CK = 128
NCHUNK = E // CK
ITERS = (NCHUNK + NW - 1) // NW
NP = 10240
ZCH = NP // CK // NS

_SC_MESH = plsc.VectorSubcoreMesh(
    core_axis_name="c", subcore_axis_name="s", num_cores=NC, num_subcores=NS
)


def _silu(x):
    return x * jax.nn.sigmoid(x)


def _make_gather(d):
    @functools.partial(
        pl.kernel,
        out_type=(
            jax.ShapeDtypeStruct((E, d), jnp.float32),
            jax.ShapeDtypeStruct((E, d), jnp.float32),
        ),
        mesh=_SC_MESH,
        scratch_types=[
            [pltpu.VMEM((CK,), jnp.int32)] * 2,
            [pltpu.VMEM((CK,), jnp.int32)] * 2,
            [pltpu.VMEM((CK, d), jnp.float32)] * 2,
            [pltpu.VMEM((CK, d), jnp.float32)] * 2,
            [pltpu.SemaphoreType.DMA] * 2,
            [pltpu.SemaphoreType.DMA] * 2,
        ],
    )
    def gather(table, src, dst, out_a, out_b, idx_a, idx_b, rows_a, rows_b,
               gsem, wsem):
        wid = lax.axis_index("s") * NC + lax.axis_index("c")

        def valid(t):
            return (t >= 0) & (t < ITERS) & (t * NW + wid < NCHUNK)

        def bslice(t):
            return pl.ds((t * NW + wid) * CK, CK)

        @pl.loop(0, ITERS + 1)
        def _(t):
            for s in (0, 1):
                @pl.when(t % 2 == s)
                def _():
                    @pl.when(valid(t - 2))
                    def _():
                        pltpu.make_async_copy(
                            rows_a[s], out_a.at[bslice(t - 2)], wsem[s]).wait()
                        pltpu.make_async_copy(
                            rows_b[s], out_b.at[bslice(t - 2)], wsem[s]).wait()

                    @pl.when(valid(t))
                    def _():
                        pltpu.sync_copy(src.at[bslice(t)], idx_a[s])
                        pltpu.sync_copy(dst.at[bslice(t)], idx_b[s])
                        pltpu.async_copy(table.at[idx_a[s]], rows_a[s], gsem[s])
                        pltpu.async_copy(table.at[idx_b[s]], rows_b[s], gsem[s])

            for o in (0, 1):
                @pl.when((t - 1) % 2 == o)
                def _():
                    @pl.when(valid(t - 1))
                    def _():
                        pltpu.make_async_copy(
                            table.at[idx_a[o]], rows_a[o], gsem[o]).wait()
                        pltpu.make_async_copy(
                            table.at[idx_b[o]], rows_b[o], gsem[o]).wait()
                        pltpu.async_copy(rows_a[o], out_a.at[bslice(t - 1)],
                                         wsem[o])
                        pltpu.async_copy(rows_b[o], out_b.at[bslice(t - 1)],
                                         wsem[o])

        sl = (ITERS - 1) % 2

        @pl.when(valid(ITERS - 1))
        def _():
            pltpu.make_async_copy(
                rows_a[sl], out_a.at[bslice(ITERS - 1)], wsem[sl]).wait()
            pltpu.make_async_copy(
                rows_b[sl], out_b.at[bslice(ITERS - 1)], wsem[sl]).wait()

    return gather


_gather_h = _make_gather(F)


_SCHUNKS = NCHUNK // NC
_SITERS = (_SCHUNKS + NS - 1) // NS


@functools.partial(
    pl.kernel,
    out_type=jax.ShapeDtypeStruct((NC * NP, F), jnp.float32),
    mesh=_SC_MESH,
    scratch_types=[
        [pltpu.VMEM((CK,), jnp.int32)] * 2,
        [pltpu.VMEM((CK, F), jnp.float32)] * 2,
        [pltpu.VMEM((CK * FM // F, F), jnp.float32)] * 2,
        [pltpu.SemaphoreType.DMA] * 2,
        [pltpu.SemaphoreType.DMA] * 2,
        pltpu.SemaphoreType.DMA,
        pltpu.VMEM_SHARED((NP, F), jnp.float32),
    ],
)
def _scatter_add(m, dst, iota, out, idx_v, m_v, n_v, msem, ssem, sem, acc_sh):
    cid = lax.axis_index("c")
    sid = lax.axis_index("s")

    for s in (0, 1):
        @pl.loop(0, CK)
        def _(r):
            @pl.loop(0, F // 16)
            def _(k):
                m_v[s][r, pl.ds(k * 16, 16)] = jnp.zeros((16,), jnp.float32)

    @pl.loop(0, ZCH)
    def _(j):
        c = sid * ZCH + j
        pltpu.sync_copy(iota.at[pl.ds(c * CK, CK)], idx_v[0])
        pltpu.sync_copy(m_v[0], acc_sh.at[idx_v[0]])

    plsc.subcore_barrier()

    def valid(t):
        return (t >= 0) & (t < _SITERS) & (t * NS + sid < _SCHUNKS)

    def bslice(t):
        return pl.ds(cid * (E // NC) + (t * NS + sid) * CK, CK)

    _PR = CK * FM // F

    def mslice(t):
        base = cid * (E // NC // (F // FM)) + (t * NS + sid) * _PR
        return pl.ds(pl.multiple_of(base, 8), _PR)

    @pl.loop(0, _SITERS + 1)
    def _(t):
        for s in (0, 1):
            @pl.when(t % 2 == s)
            def _():
                @pl.when(valid(t - 2))
                def _():
                    pltpu.make_async_copy(
                        m_v[s], acc_sh.at[idx_v[s]], ssem[s]).wait()

                @pl.when(valid(t))
                def _():
                    pltpu.sync_copy(dst.at[bslice(t)], idx_v[s])
                    pltpu.async_copy(m.at[mslice(t)], n_v[s], msem[s])

        for o in (0, 1):
            @pl.when((t - 1) % 2 == o)
            def _():
                @pl.when(valid(t - 1))
                def _():
                    pltpu.make_async_copy(
                        m.at[mslice(t - 1)], n_v[o], msem[o]).wait()

                    @pl.loop(0, _PR)
                    def _(r):
                        for k in range(F // FM):
                            m_v[o][r * (F // FM) + k, pl.ds(0, FM)] = \
                                n_v[o][r, pl.ds(k * FM, FM)]

                    pltpu.async_copy(m_v[o], acc_sh.at[idx_v[o]], ssem[o],
                                     add=True)

    sl = (_SITERS - 1) % 2

    @pl.when(valid(_SITERS - 1))
    def _():
        pltpu.make_async_copy(m_v[sl], acc_sh.at[idx_v[sl]], ssem[sl]).wait()

    plsc.subcore_barrier()

    @pl.loop(0, ZCH)
    def _(j):
        c = sid * ZCH + j
        pltpu.sync_copy(iota.at[pl.ds(c * CK, CK)], idx_v[0])
        pltpu.async_copy(acc_sh.at[idx_v[0]], m_v[0], sem).wait()
        pltpu.sync_copy(m_v[0], out.at[pl.ds(cid * NP + c * CK, CK)])


_BN = 2000
_BE = 4000


def _h0_body(z_ref, tab_ref, o_ref):
    z = z_ref[...]
    oh = (z == lax.broadcasted_iota(jnp.int32, (_BN, 16), 1)).astype(jnp.float32)
    o_ref[...] = jnp.dot(oh, tab_ref[...], preferred_element_type=jnp.float32)


def _norm_body(ps_ref, pd_ref, o_ref):
    d = ps_ref[...] - pd_ref[...]
    o_ref[...] = jnp.sum(d * d, axis=1, keepdims=True)


def _edge_body(xi_ref, xj_ref, n2_ref, w1i_ref, w1j_ref, wc_ref, b1_ref,
               w2_ref, b2_ref, o_ref):
    pre = (
        jnp.dot(xi_ref[...], w1i_ref[...], preferred_element_type=jnp.float32)
        + jnp.dot(xj_ref[...], w1j_ref[...], preferred_element_type=jnp.float32)
        + n2_ref[...] * wc_ref[...]
        + b1_ref[...]
    )
    m1 = _silu(pre)
    o_ref[...] = _silu(
        jnp.dot(m1, w2_ref[...], preferred_element_type=jnp.float32) + b2_ref[...]
    )


def _upd_body(a0_ref, a1_ref, h_ref, wua_ref, wuh_ref, b1_ref, w2_ref, b2_ref,
              o_ref):
    agg = a0_ref[...] + a1_ref[...]
    h = h_ref[...]
    t = _silu(
        jnp.dot(agg, wua_ref[...], preferred_element_type=jnp.float32)
        + jnp.dot(h, wuh_ref[...], preferred_element_type=jnp.float32)
        + b1_ref[...]
    )
    o_ref[...] = jnp.dot(t, w2_ref[...], preferred_element_type=jnp.float32) \
        + b2_ref[...] + h


def _final_body(h_ref, b_ref, wp1_ref, bp1_ref, wp2_ref, bp2_ref,
                wr1_ref, br1_ref, wr2_ref, br2_ref, o_ref, acc_ref):
    i = pl.program_id(0)

    @pl.when(i == 0)
    def _():
        acc_ref[...] = jnp.zeros_like(acc_ref)

    proj = jnp.dot(
        _silu(jnp.dot(h_ref[...], wp1_ref[...], preferred_element_type=jnp.float32)
              + bp1_ref[...]),
        wp2_ref[...], preferred_element_type=jnp.float32) + bp2_ref[...]
    oh = (b_ref[...] == lax.broadcasted_iota(jnp.int32, (_BN, NG), 1)).astype(
        jnp.float32)
    acc_ref[...] += lax.dot_general(
        oh, proj, (((0,), (0,)), ((), ())), preferred_element_type=jnp.float32)

    @pl.when(i == pl.num_programs(0) - 1)
    def _():
        pooled = acc_ref[...]
        o_ref[...] = jnp.dot(
            _silu(jnp.dot(pooled, wr1_ref[...], preferred_element_type=jnp.float32)
                  + br1_ref[...]),
            wr2_ref[...], preferred_element_type=jnp.float32) + br2_ref[...]


def _full(shape):
    return pl.BlockSpec(shape, lambda i: tuple(0 for _ in shape))


def _rows(shape):
    return pl.BlockSpec(shape, lambda i: (i,) + tuple(0 for _ in shape[1:]))


def _tc_h0(z2d, tab16):
    return pl.pallas_call(
        _h0_body,
        grid=(N // _BN,),
        in_specs=[_rows((_BN, 1)), _full((16, F))],
        out_specs=_rows((_BN, F)),
        out_shape=jax.ShapeDtypeStruct((N, F), jnp.float32),
    )(z2d, tab16)


def _tc_norm(ps, pd):
    return pl.pallas_call(
        _norm_body,
        grid=(E // _BE,),
        in_specs=[_rows((_BE, F)), _rows((_BE, F))],
        out_specs=_rows((_BE, 1)),
        out_shape=jax.ShapeDtypeStruct((E, 1), jnp.float32),
    )(ps, pd)


def _tc_edge(xi, xj, n2, w1i, w1j, wc, b1, w2, b2):
    return pl.pallas_call(
        _edge_body,
        grid=(E // _BE,),
        in_specs=[
            _rows((_BE, F)), _rows((_BE, F)), _rows((_BE, 1)),
            _full((F, MH)), _full((F, MH)), _full((1, MH)), _full((1, MH)),
            _full((MH, FM)), _full((1, FM)),
        ],
        out_specs=_rows((_BE, FM)),
        out_shape=jax.ShapeDtypeStruct((E, FM), jnp.float32),
    )(xi, xj, n2, w1i, w1j, wc, b1, w2, b2)


def _tc_upd(a0, a1, h, wua, wuh, b1, w2, b2):
    return pl.pallas_call(
        _upd_body,
        grid=(N // _BN,),
        in_specs=[
            _rows((_BN, F)), _rows((_BN, F)), _rows((_BN, F)),
            _full((F, 2 * F)), _full((F, 2 * F)), _full((1, 2 * F)),
            _full((2 * F, F)), _full((1, F)),
        ],
        out_specs=_rows((_BN, F)),
        out_shape=jax.ShapeDtypeStruct((N, F), jnp.float32),
    )(a0, a1, h, wua, wuh, b1, w2, b2)


def _tc_final(h, b2d, wp1, bp1, wp2, bp2, wr1, br1, wr2p, br2p):
    return pl.pallas_call(
        _final_body,
        grid=(N // _BN,),
        in_specs=[
            _rows((_BN, F)), _rows((_BN, 1)),
            _full((F, H)), _full((1, H)), _full((H, H)), _full((1, H)),
            _full((H, H)), _full((1, H)), _full((H, 128)), _full((1, 128)),
        ],
        out_specs=_full((NG, 128)),
        out_shape=jax.ShapeDtypeStruct((NG, 128), jnp.float32),
        scratch_shapes=[pltpu.VMEM((NG, 128), jnp.float32)],
    )(h, b2d, wp1, bp1, wp2, bp2, wr1, br1, wr2p, br2p)


def kernel(pos, emb, Wm1, bm1, Wm2, bm2, Wu1, bu1, Wu2, bu2, Wp1, bp1,
           Wp2, bp2, Wr1, br1, Wr2, br2, z, edge_index, batch):
    atomic_number = jnp.array([-1, 0, -1, -1, -1, -1, 1, 2, 3, 4], dtype=jnp.int32)
    tab16 = jnp.zeros((16, F), jnp.float32).at[:10].set(
        emb[jnp.clip(atomic_number, 0, 4)])
    src = edge_index[0]
    dst = edge_index[1]

    pos128 = jnp.zeros((N, F), jnp.float32).at[:, :3].set(pos)
    ps, pd = _gather_h(pos128, src, dst)
    n2 = _tc_norm(ps, pd)

    h = _tc_h0(z[:, None], tab16)

    iota = jnp.arange(NP, dtype=jnp.int32)
    wuap = jnp.zeros((L, F, 2 * F), jnp.float32).at[:, :FM].set(Wu1[:, :FM])

    for l in range(L):
        xj, xi = _gather_h(h, src, dst)
        m = _tc_edge(
            xi, xj, n2,
            Wm1[l, :F], Wm1[l, F:2 * F], Wm1[l, 2 * F:], bm1[l][None, :],
            Wm2[l], bm2[l][None, :],
        )
        aggp = _scatter_add(m.reshape(E * FM // F, F), dst, iota)
        h = _tc_upd(
            aggp[:N], aggp[NP:NP + N], h,
            wuap[l], Wu1[l, FM:], bu1[l][None, :],
            Wu2[l], bu2[l][None, :],
        )

    wr2p = jnp.zeros((H, 128), jnp.float32).at[:, :T].set(Wr2)
    br2p = jnp.zeros((1, 128), jnp.float32).at[:, :T].set(br2)
    out = _tc_final(h, batch[:, None], Wp1, bp1[None, :], Wp2, bp2[None, :],
                    Wr1, br1[None, :], wr2p, br2p)
    return out[:, :T]


# speedup vs baseline: 1.1669x; 1.1669x over previous
import functools

import jax
import jax.numpy as jnp
from jax import lax
from jax.experimental import pallas as pl
from jax.experimental.pallas import tpu as pltpu
from jax.experimental.pallas import tpu_sc as plsc

N = 10000
E = 320000
L = 7
F = 128
FM = 16
H = 128
T = 12
NG = 64
MH = 2 * (2 * F + 1)

NC = 2
NS = 16
NW = NC * NS
CK = 128
NCHUNK = E // CK
ITERS = (NCHUNK + NW - 1) // NW
NP = 10240
ZCH = NP // CK // NS

_SC_MESH = plsc.VectorSubcoreMesh(
    core_axis_name="c", subcore_axis_name="s", num_cores=NC, num_subcores=NS
)


def _silu(x):
    return x * jax.nn.sigmoid(x)


def _make_gather(d):
    @functools.partial(
        pl.kernel,
        out_type=(
            jax.ShapeDtypeStruct((E, d), jnp.float32),
            jax.ShapeDtypeStruct((E, d), jnp.float32),
        ),
        mesh=_SC_MESH,
        scratch_types=[
            [pltpu.VMEM((2 * CK,), jnp.int32)] * 2,
            [pltpu.VMEM((CK, d), jnp.float32)] * 2,
            [pltpu.VMEM((CK, d), jnp.float32)] * 2,
            [pltpu.SemaphoreType.DMA] * 2,
            [pltpu.SemaphoreType.DMA] * 2,
        ],
    )
    def gather(table, idx2, out_a, out_b, idx_ab, rows_a, rows_b, gsem, wsem):
        wid = lax.axis_index("s") * NC + lax.axis_index("c")

        def valid(t):
            return (t >= 0) & (t < ITERS) & (t * NW + wid < NCHUNK)

        def bslice(t):
            return pl.ds((t * NW + wid) * CK, CK)

        @pl.loop(0, ITERS + 1)
        def _(t):
            for s in (0, 1):
                ia = idx_ab[s].at[pl.ds(0, CK)]
                ib = idx_ab[s].at[pl.ds(CK, CK)]

                @pl.when(t % 2 == s)
                def _():
                    @pl.when(valid(t - 2))
                    def _():
                        pltpu.make_async_copy(
                            rows_a[s], out_a.at[bslice(t - 2)], wsem[s]).wait()
                        pltpu.make_async_copy(
                            rows_b[s], out_b.at[bslice(t - 2)], wsem[s]).wait()

                    @pl.when(valid(t))
                    def _():
                        pltpu.sync_copy(
                            idx2.at[pl.ds((t * NW + wid) * 2 * CK, 2 * CK)],
                            idx_ab[s])
                        pltpu.async_copy(table.at[ia], rows_a[s], gsem[s])
                        pltpu.async_copy(table.at[ib], rows_b[s], gsem[s])

            for o in (0, 1):
                ia = idx_ab[o].at[pl.ds(0, CK)]
                ib = idx_ab[o].at[pl.ds(CK, CK)]

                @pl.when((t - 1) % 2 == o)
                def _():
                    @pl.when(valid(t - 1))
                    def _():
                        pltpu.make_async_copy(
                            table.at[ia], rows_a[o], gsem[o]).wait()
                        pltpu.make_async_copy(
                            table.at[ib], rows_b[o], gsem[o]).wait()
                        pltpu.async_copy(rows_a[o], out_a.at[bslice(t - 1)],
                                         wsem[o])
                        pltpu.async_copy(rows_b[o], out_b.at[bslice(t - 1)],
                                         wsem[o])

        sl = (ITERS - 1) % 2

        @pl.when(valid(ITERS - 1))
        def _():
            pltpu.make_async_copy(
                rows_a[sl], out_a.at[bslice(ITERS - 1)], wsem[sl]).wait()
            pltpu.make_async_copy(
                rows_b[sl], out_b.at[bslice(ITERS - 1)], wsem[sl]).wait()

    return gather


_gather_h = _make_gather(F)


_SCHUNKS = NCHUNK // NC
_SITERS = (_SCHUNKS + NS - 1) // NS


@functools.partial(
    pl.kernel,
    out_type=jax.ShapeDtypeStruct((NC * NP, F), jnp.float32),
    mesh=_SC_MESH,
    scratch_types=[
        [pltpu.VMEM((CK,), jnp.int32)] * 2,
        [pltpu.VMEM((CK, F), jnp.float32)] * 2,
        [pltpu.SemaphoreType.DMA] * 2,
        [pltpu.SemaphoreType.DMA] * 2,
        pltpu.SemaphoreType.DMA,
        pltpu.VMEM_SHARED((NP, F), jnp.float32),
    ],
)
def _scatter_add(m, dst, iota, out, idx_v, m_v, msem, ssem, sem, acc_sh):
    cid = lax.axis_index("c")
    sid = lax.axis_index("s")

    @pl.loop(0, CK)
    def _(r):
        @pl.loop(0, F // 16)
        def _(k):
            m_v[0][r, pl.ds(k * 16, 16)] = jnp.zeros((16,), jnp.float32)

    @pl.loop(0, ZCH)
    def _(j):
        c = sid * ZCH + j
        pltpu.sync_copy(iota.at[pl.ds(c * CK, CK)], idx_v[0])
        pltpu.sync_copy(m_v[0], acc_sh.at[idx_v[0]])

    plsc.subcore_barrier()

    def valid(t):
        return (t >= 0) & (t < _SITERS) & (t * NS + sid < _SCHUNKS)

    def bslice(t):
        return pl.ds(cid * (E // NC) + (t * NS + sid) * CK, CK)


    @pl.loop(0, _SITERS + 1)
    def _(t):
        for s in (0, 1):
            @pl.when(t % 2 == s)
            def _():
                @pl.when(valid(t - 2))
                def _():
                    pltpu.make_async_copy(
                        m_v[s], acc_sh.at[idx_v[s]], ssem[s]).wait()

                @pl.when(valid(t))
                def _():
                    pltpu.sync_copy(dst.at[bslice(t)], idx_v[s])
                    pltpu.async_copy(m.at[bslice(t)], m_v[s], msem[s])

        for o in (0, 1):
            @pl.when((t - 1) % 2 == o)
            def _():
                @pl.when(valid(t - 1))
                def _():
                    pltpu.make_async_copy(
                        m.at[bslice(t - 1)], m_v[o], msem[o]).wait()
                    pltpu.async_copy(m_v[o], acc_sh.at[idx_v[o]], ssem[o],
                                     add=True)

    sl = (_SITERS - 1) % 2

    @pl.when(valid(_SITERS - 1))
    def _():
        pltpu.make_async_copy(m_v[sl], acc_sh.at[idx_v[sl]], ssem[sl]).wait()

    plsc.subcore_barrier()

    @pl.loop(0, ZCH)
    def _(j):
        c = sid * ZCH + j
        pltpu.sync_copy(iota.at[pl.ds(c * CK, CK)], idx_v[0])
        pltpu.async_copy(acc_sh.at[idx_v[0]], m_v[0], sem).wait()
        pltpu.sync_copy(m_v[0], out.at[pl.ds(cid * NP + c * CK, CK)])


_BN = 2000
_BE = 4000


def _h0_body(z_ref, tab_ref, o_ref):
    z = z_ref[...]
    oh = (z == lax.broadcasted_iota(jnp.int32, (_BN, 16), 1)).astype(jnp.float32)
    o_ref[...] = jnp.dot(oh, tab_ref[...], preferred_element_type=jnp.float32)


def _norm_body(ps_ref, pd_ref, o_ref):
    d = ps_ref[...] - pd_ref[...]
    o_ref[...] = jnp.sum(d * d, axis=1, keepdims=True)


def _edge_body(xi_ref, xj_ref, n2_ref, w1i_ref, w1j_ref, wc_ref, b1_ref,
               w2_ref, b2_ref, o_ref):
    pre = (
        jnp.dot(xi_ref[...], w1i_ref[...], preferred_element_type=jnp.float32)
        + jnp.dot(xj_ref[...], w1j_ref[...], preferred_element_type=jnp.float32)
        + n2_ref[...] * wc_ref[...]
        + b1_ref[...]
    )
    m1 = _silu(pre)
    o_ref[...] = _silu(
        jnp.dot(m1, w2_ref[...], preferred_element_type=jnp.float32) + b2_ref[...]
    )


def _upd_body(a0_ref, a1_ref, h_ref, wua_ref, wuh_ref, b1_ref, w2_ref, b2_ref,
              o_ref):
    agg = a0_ref[...] + a1_ref[...]
    h = h_ref[...]
    t = _silu(
        jnp.dot(agg, wua_ref[...], preferred_element_type=jnp.float32)
        + jnp.dot(h, wuh_ref[...], preferred_element_type=jnp.float32)
        + b1_ref[...]
    )
    o_ref[...] = jnp.dot(t, w2_ref[...], preferred_element_type=jnp.float32) \
        + b2_ref[...] + h


def _final_body(h_ref, b_ref, wp1_ref, bp1_ref, wp2_ref, bp2_ref,
                wr1_ref, br1_ref, wr2_ref, br2_ref, o_ref, acc_ref):
    i = pl.program_id(0)

    @pl.when(i == 0)
    def _():
        acc_ref[...] = jnp.zeros_like(acc_ref)

    proj = jnp.dot(
        _silu(jnp.dot(h_ref[...], wp1_ref[...], preferred_element_type=jnp.float32)
              + bp1_ref[...]),
        wp2_ref[...], preferred_element_type=jnp.float32) + bp2_ref[...]
    oh = (b_ref[...] == lax.broadcasted_iota(jnp.int32, (_BN, NG), 1)).astype(
        jnp.float32)
    acc_ref[...] += lax.dot_general(
        oh, proj, (((0,), (0,)), ((), ())), preferred_element_type=jnp.float32)

    @pl.when(i == pl.num_programs(0) - 1)
    def _():
        pooled = acc_ref[...]
        o_ref[...] = jnp.dot(
            _silu(jnp.dot(pooled, wr1_ref[...], preferred_element_type=jnp.float32)
                  + br1_ref[...]),
            wr2_ref[...], preferred_element_type=jnp.float32) + br2_ref[...]


def _full(shape):
    return pl.BlockSpec(shape, lambda i: tuple(0 for _ in shape))


def _rows(shape):
    return pl.BlockSpec(shape, lambda i: (i,) + tuple(0 for _ in shape[1:]))


def _tc_h0(z2d, tab16):
    return pl.pallas_call(
        _h0_body,
        grid=(N // _BN,),
        in_specs=[_rows((_BN, 1)), _full((16, F))],
        out_specs=_rows((_BN, F)),
        out_shape=jax.ShapeDtypeStruct((N, F), jnp.float32),
    )(z2d, tab16)


def _tc_norm(ps, pd):
    return pl.pallas_call(
        _norm_body,
        grid=(E // _BE,),
        in_specs=[_rows((_BE, F)), _rows((_BE, F))],
        out_specs=_rows((_BE, 1)),
        out_shape=jax.ShapeDtypeStruct((E, 1), jnp.float32),
    )(ps, pd)


def _tc_edge(xi, xj, n2, w1i, w1j, wc, b1, w2, b2):
    return pl.pallas_call(
        _edge_body,
        grid=(E // _BE,),
        in_specs=[
            _rows((_BE, F)), _rows((_BE, F)), _rows((_BE, 1)),
            _full((F, MH)), _full((F, MH)), _full((1, MH)), _full((1, MH)),
            _full((MH, F)), _full((1, F)),
        ],
        out_specs=_rows((_BE, F)),
        out_shape=jax.ShapeDtypeStruct((E, F), jnp.float32),
    )(xi, xj, n2, w1i, w1j, wc, b1, w2, b2)


def _tc_upd(a0, a1, h, wua, wuh, b1, w2, b2):
    return pl.pallas_call(
        _upd_body,
        grid=(N // _BN,),
        in_specs=[
            _rows((_BN, F)), _rows((_BN, F)), _rows((_BN, F)),
            _full((F, 2 * F)), _full((F, 2 * F)), _full((1, 2 * F)),
            _full((2 * F, F)), _full((1, F)),
        ],
        out_specs=_rows((_BN, F)),
        out_shape=jax.ShapeDtypeStruct((N, F), jnp.float32),
    )(a0, a1, h, wua, wuh, b1, w2, b2)


def _tc_final(h, b2d, wp1, bp1, wp2, bp2, wr1, br1, wr2p, br2p):
    return pl.pallas_call(
        _final_body,
        grid=(N // _BN,),
        in_specs=[
            _rows((_BN, F)), _rows((_BN, 1)),
            _full((F, H)), _full((1, H)), _full((H, H)), _full((1, H)),
            _full((H, H)), _full((1, H)), _full((H, 128)), _full((1, 128)),
        ],
        out_specs=_full((NG, 128)),
        out_shape=jax.ShapeDtypeStruct((NG, 128), jnp.float32),
        scratch_shapes=[pltpu.VMEM((NG, 128), jnp.float32)],
    )(h, b2d, wp1, bp1, wp2, bp2, wr1, br1, wr2p, br2p)


def kernel(pos, emb, Wm1, bm1, Wm2, bm2, Wu1, bu1, Wu2, bu2, Wp1, bp1,
           Wp2, bp2, Wr1, br1, Wr2, br2, z, edge_index, batch):
    atomic_number = jnp.array([-1, 0, -1, -1, -1, -1, 1, 2, 3, 4], dtype=jnp.int32)
    tab16 = jnp.zeros((16, F), jnp.float32).at[:10].set(
        emb[jnp.clip(atomic_number, 0, 4)])
    src = edge_index[0]
    dst = edge_index[1]

    idx2 = jnp.stack(
        [src.reshape(NCHUNK, CK), dst.reshape(NCHUNK, CK)], axis=1
    ).reshape(-1)

    pos128 = jnp.zeros((N, F), jnp.float32).at[:, :3].set(pos)
    ps, pd = _gather_h(pos128, idx2)
    n2 = _tc_norm(ps, pd)

    h = _tc_h0(z[:, None], tab16)

    iota = jnp.arange(NP, dtype=jnp.int32)
    wm2p = jnp.zeros((L, MH, F), jnp.float32).at[:, :, :FM].set(Wm2)
    bm2p = jnp.zeros((L, F), jnp.float32).at[:, :FM].set(bm2)
    wuap = jnp.zeros((L, F, 2 * F), jnp.float32).at[:, :FM].set(Wu1[:, :FM])

    for l in range(L):
        xj, xi = _gather_h(h, idx2)
        m = _tc_edge(
            xi, xj, n2,
            Wm1[l, :F], Wm1[l, F:2 * F], Wm1[l, 2 * F:], bm1[l][None, :],
            wm2p[l], bm2p[l][None, :],
        )
        aggp = _scatter_add(m, dst, iota)
        h = _tc_upd(
            aggp[:N], aggp[NP:NP + N], h,
            wuap[l], Wu1[l, FM:], bu1[l][None, :],
            Wu2[l], bu2[l][None, :],
        )

    wr2p = jnp.zeros((H, 128), jnp.float32).at[:, :T].set(Wr2)
    br2p = jnp.zeros((1, 128), jnp.float32).at[:, :T].set(br2)
    out = _tc_final(h, batch[:, None], Wp1, bp1[None, :], Wp2, bp2[None, :],
                    Wr1, br1[None, :], wr2p, br2p)
    return out[:, :T]
